# baseline (speedup 1.0000x reference)
import jax
import jax.numpy as jnp
from jax import lax
from jax.experimental import pallas as pl
from jax.experimental.pallas import tpu as pltpu
from jax.experimental.pallas import tpu_sc as plsc

_T = 4096
_B = 16
_C = 256
_NW = 32
_ROWS_W = _T // _NW
_CH = 4
_NCH = _ROWS_W // _CH
_LANES = 16
_NCOL = _B * _C // _LANES
_CPB = _C // _LANES
_R = 1.0 / (2 * _T - 1)


def _body(y_hbm, out_hbm, in_v, out_v, sin, sout):
    c = lax.axis_index("c")
    s = lax.axis_index("s")
    wid = s * 2 + c
    base = wid * _ROWS_W

    def issue_in(ci):
        b = ci % 2
        row0 = base + ci * _CH
        prev = pltpu.async_copy(y_hbm.at[pl.ds(jnp.maximum(row0 - 1, 0), 1)],
                                in_v[b].at[pl.ds(0, 1)], sin[b])
        if ci < _NCH - 1:
            return (
                prev,
                pltpu.async_copy(y_hbm.at[pl.ds(row0, _CH + 1)],
                                 in_v[b].at[pl.ds(1, _CH + 1)], sin[b]),
            )
        return (
            prev,
            pltpu.async_copy(y_hbm.at[pl.ds(row0, _CH)],
                             in_v[b].at[pl.ds(1, _CH)], sin[b]),
            pltpu.async_copy(y_hbm.at[pl.ds(jnp.minimum(row0 + _CH, _T - 1), 1)],
                             in_v[b].at[pl.ds(_CH + 1, 1)], sin[b]),
        )

    def issue_out(ci):
        b = ci % 2
        row0 = base + ci * _CH
        return pltpu.async_copy(out_v[b], out_hbm.at[pl.ds(2 * row0, 2 * _CH)],
                                sout[b])

    def compute(ci):
        b = ci % 2
        iv, ov = in_v[b], out_v[b]
        row0_f = (base + ci * _CH).astype(jnp.float32)
        coeffs = []
        for l in range(_CH):
            mf = row0_f + float(l)
            coeffs.append((mf * _R, (float(_T - 1) - mf) * _R))

        @plsc.parallel_loop(0, _NCOL, 1, unroll=4)
        def col(j):
            sub = j // _CPB
            sl = pl.ds((j % _CPB) * _LANES, _LANES)
            vals = [iv[l, sub, sl] for l in range(_CH + 2)]
            diffs = [vals[l + 1] - vals[l] for l in range(_CH + 1)]
            for l in range(_CH):
                a, bb = coeffs[l]
                y0 = vals[l + 1]
                ov[2 * l, sub, sl] = y0 - a * diffs[l]
                ov[2 * l + 1, sub, sl] = y0 + bb * diffs[l + 1]

    hin = {}
    hout = {}
    hin[0] = issue_in(0)
    for ci in range(_NCH):
        if ci + 1 < _NCH:
            hin[ci + 1] = issue_in(ci + 1)
        for h in hin.pop(ci):
            h.wait()
        if ci >= 2:
            hout.pop(ci - 2).wait()
        compute(ci)
        hout[ci] = issue_out(ci)
    hout.pop(_NCH - 2).wait()
    hout.pop(_NCH - 1).wait()


@jax.jit
def kernel(y):
    T, B, C = y.shape
    call = pl.kernel(
        _body,
        out_type=jax.ShapeDtypeStruct((2 * T, B, C), jnp.float32),
        mesh=plsc.VectorSubcoreMesh(core_axis_name="c", subcore_axis_name="s"),
        scratch_types=[
            [pltpu.VMEM((_CH + 2, _B, _C), jnp.float32) for _ in range(2)],
            [pltpu.VMEM((2 * _CH, _B, _C), jnp.float32) for _ in range(2)],
            [pltpu.SemaphoreType.DMA for _ in range(2)],
            [pltpu.SemaphoreType.DMA for _ in range(2)],
        ],
    )
    return call(y)


# speedup vs baseline: 1.0200x; 1.0200x over previous
import jax
import jax.numpy as jnp
from jax import lax
from jax.experimental import pallas as pl
from jax.experimental.pallas import tpu as pltpu
from jax.experimental.pallas import tpu_sc as plsc

_T = 4096
_B = 16
_C = 256
_NW = 32
_ROWS_W = _T // _NW
_CH = 4
_NCH = _ROWS_W // _CH
_LANES = 16
_NCOL = _B * _C // _LANES
_CPB = _C // _LANES
_R = 1.0 / (2 * _T - 1)


def _body(y_hbm, out_hbm, in_v, out_v, sin, sout):
    c = lax.axis_index("c")
    s = lax.axis_index("s")
    wid = s * 2 + c
    base = wid * _ROWS_W

    def issue_in(ci):
        b = ci % 2
        row0 = base + ci * _CH
        prev = pltpu.async_copy(y_hbm.at[pl.ds(jnp.maximum(row0 - 1, 0), 1)],
                                in_v[b].at[pl.ds(0, 1)], sin[b])
        if ci < _NCH - 1:
            return (
                prev,
                pltpu.async_copy(y_hbm.at[pl.ds(row0, _CH + 1)],
                                 in_v[b].at[pl.ds(1, _CH + 1)], sin[b]),
            )
        return (
            prev,
            pltpu.async_copy(y_hbm.at[pl.ds(row0, _CH)],
                             in_v[b].at[pl.ds(1, _CH)], sin[b]),
            pltpu.async_copy(y_hbm.at[pl.ds(jnp.minimum(row0 + _CH, _T - 1), 1)],
                             in_v[b].at[pl.ds(_CH + 1, 1)], sin[b]),
        )

    def issue_out(ci):
        b = ci % 2
        row0 = base + ci * _CH
        return pltpu.async_copy(out_v[b], out_hbm.at[pl.ds(2 * row0, 2 * _CH)],
                                sout[b])

    def compute(ci):
        b = ci % 2
        iv, ov = in_v[b], out_v[b]
        row0_f = (base + ci * _CH).astype(jnp.float32)
        coeffs = []
        for l in range(_CH):
            mf = row0_f + float(l)
            coeffs.append((mf * _R, (float(_T - 1) - mf) * _R))

        @plsc.parallel_loop(0, _NCOL, 1, unroll=2)
        def col(j):
            sub = j // _CPB
            sl = pl.ds((j % _CPB) * _LANES, _LANES)
            vals = [iv[l, sub, sl] for l in range(_CH + 2)]
            diffs = [vals[l + 1] - vals[l] for l in range(_CH + 1)]
            for l in range(_CH):
                a, bb = coeffs[l]
                y0 = vals[l + 1]
                ov[2 * l, sub, sl] = y0 - a * diffs[l]
                ov[2 * l + 1, sub, sl] = y0 + bb * diffs[l + 1]

    hin = {}
    hout = {}
    hin[0] = issue_in(0)
    for ci in range(_NCH):
        if ci + 1 < _NCH:
            hin[ci + 1] = issue_in(ci + 1)
        for h in hin.pop(ci):
            h.wait()
        if ci >= 2:
            hout.pop(ci - 2).wait()
        compute(ci)
        hout[ci] = issue_out(ci)
    hout.pop(_NCH - 2).wait()
    hout.pop(_NCH - 1).wait()


@jax.jit
def kernel(y):
    T, B, C = y.shape
    call = pl.kernel(
        _body,
        out_type=jax.ShapeDtypeStruct((2 * T, B, C), jnp.float32),
        mesh=plsc.VectorSubcoreMesh(core_axis_name="c", subcore_axis_name="s"),
        scratch_types=[
            [pltpu.VMEM((_CH + 2, _B, _C), jnp.float32) for _ in range(2)],
            [pltpu.VMEM((2 * _CH, _B, _C), jnp.float32) for _ in range(2)],
            [pltpu.SemaphoreType.DMA for _ in range(2)],
            [pltpu.SemaphoreType.DMA for _ in range(2)],
        ],
    )
    return call(y)


# speedup vs baseline: 1.0772x; 1.0562x over previous
import jax
import jax.numpy as jnp
from jax import lax
from jax.experimental import pallas as pl
from jax.experimental.pallas import tpu as pltpu
from jax.experimental.pallas import tpu_sc as plsc

_T = 4096
_B = 16
_C = 256
_NW = 32
_ROWS_W = _T // _NW
_CH = 4
_NCH = _ROWS_W // _CH
_LANES = 16
_CPB = _C // _LANES
_R = 1.0 / (2 * _T - 1)


def _body(y_hbm, out_hbm, in_v, out_v, sin, sout):
    c = lax.axis_index("c")
    s = lax.axis_index("s")
    wid = s * 2 + c
    base = wid * _ROWS_W

    def issue_in_first(ci):
        b = ci % 2
        row0 = base + ci * _CH
        pltpu.async_copy(y_hbm.at[pl.ds(jnp.maximum(row0 - 1, 0), 1)],
                         in_v[b].at[pl.ds(0, 1)], sin[b])
        pltpu.async_copy(y_hbm.at[pl.ds(row0, _CH + 1)],
                         in_v[b].at[pl.ds(1, _CH + 1)], sin[b])

    def issue_in_dyn(row0, b):
        pltpu.async_copy(y_hbm.at[pl.ds(row0 - 1, 1)],
                         in_v[b].at[pl.ds(0, 1)], sin[b])

        @pl.when(row0 + _CH <= _T - 1)
        def _():
            pltpu.async_copy(y_hbm.at[pl.ds(row0, _CH + 1)],
                             in_v[b].at[pl.ds(1, _CH + 1)], sin[b])

        @pl.when(row0 + _CH > _T - 1)
        def _():
            pltpu.async_copy(y_hbm.at[pl.ds(row0, _CH)],
                             in_v[b].at[pl.ds(1, _CH)], sin[b])
            pltpu.async_copy(y_hbm.at[pl.ds(_T - 1, 1)],
                             in_v[b].at[pl.ds(_CH + 1, 1)], sin[b])

    def wait_in(b):
        pltpu.make_async_copy(y_hbm.at[pl.ds(0, _CH + 2)], in_v[b],
                              sin[b]).wait()

    def wait_out(b):
        pltpu.make_async_copy(out_v[b], out_hbm.at[pl.ds(0, 2 * _CH)],
                              sout[b]).wait()

    def compute(row0, b):
        iv, ov = in_v[b], out_v[b]
        row0_f = row0.astype(jnp.float32)
        coeffs = []
        for l in range(_CH):
            mf = row0_f + float(l)
            coeffs.append((mf * _R, (float(_T - 1) - mf) * _R))

        @plsc.parallel_loop(0, _B, 1)
        def subloop(sub):
            for k in range(_CPB):
                sl = pl.ds(k * _LANES, _LANES)
                vals = [iv[l, sub, sl] for l in range(_CH + 2)]
                diffs = [vals[l + 1] - vals[l] for l in range(_CH + 1)]
                for l in range(_CH):
                    a, bb = coeffs[l]
                    y0 = vals[l + 1]
                    ov[2 * l, sub, sl] = y0 - a * diffs[l]
                    ov[2 * l + 1, sub, sl] = y0 + bb * diffs[l + 1]

    issue_in_first(0)
    issue_in_first(1)

    @pl.loop(0, _NCH // 2)
    def g_loop(g):
        for b in range(2):
            ci = 2 * g + b
            row0 = base + ci * _CH
            wait_in(b)

            @pl.when(g > 0)
            def _():
                wait_out(b)

            compute(row0, b)
            pltpu.async_copy(out_v[b], out_hbm.at[pl.ds(2 * row0, 2 * _CH)],
                             sout[b])

            @pl.when(ci + 2 <= _NCH - 1)
            def _():
                issue_in_dyn(row0 + 2 * _CH, b)

    wait_out(0)
    wait_out(1)


@jax.jit
def kernel(y):
    T, B, C = y.shape
    call = pl.kernel(
        _body,
        out_type=jax.ShapeDtypeStruct((2 * T, B, C), jnp.float32),
        mesh=plsc.VectorSubcoreMesh(core_axis_name="c", subcore_axis_name="s"),
        scratch_types=[
            [pltpu.VMEM((_CH + 2, _B, _C), jnp.float32) for _ in range(2)],
            [pltpu.VMEM((2 * _CH, _B, _C), jnp.float32) for _ in range(2)],
            [pltpu.SemaphoreType.DMA for _ in range(2)],
            [pltpu.SemaphoreType.DMA for _ in range(2)],
        ],
    )
    return call(y)
